# Initial kernel scaffold; baseline (speedup 1.0000x reference)
import jax
import jax.numpy as jnp
from jax.experimental import pallas as pl


def kernel(edge_attr, W0, W1, W2):
    raise NotImplementedError("write your pallas kernel here")


# speedup vs baseline: 7.4625x; 7.4625x over previous
import functools

import numpy as np
import jax
import jax.numpy as jnp
from jax import lax
from jax.experimental import pallas as pl
from jax.experimental.pallas import tpu as pltpu
from jax.experimental.pallas import tpu_sc as plsc

EMB = 128
T0, T1, T2 = 15, 16, 12
NROWS = T0 * T1 * T2
E_TOTAL = 320000
NC, NS = 2, 16
NW = NC * NS
E_PER_W = E_TOTAL // NW
CHUNK = 80
NCHUNK = E_PER_W // CHUNK

_r = np.arange(NROWS)
_M = np.zeros((NROWS, 128), np.float32)
_M[_r, _r // (T1 * T2)] = 1.0
_M[_r, T0 + (_r // T2) % T1] = 1.0
_M[_r, T0 + T1 + _r % T2] = 1.0


def _fuse_body(m_ref, w_ref, out_ref):
    out_ref[...] = jnp.dot(m_ref[...], w_ref[...],
                           preferred_element_type=jnp.float32)


def _build_fused_table(W0, W1, W2):
    wcat = jnp.zeros((128, EMB), jnp.float32)
    wcat = lax.dynamic_update_slice(wcat, W0, (0, 0))
    wcat = lax.dynamic_update_slice(wcat, W1, (T0, 0))
    wcat = lax.dynamic_update_slice(wcat, W2, (T0 + T1, 0))
    return pl.pallas_call(
        _fuse_body,
        out_shape=jax.ShapeDtypeStruct((NROWS, EMB), jnp.float32),
    )(jnp.asarray(_M), wcat)


def _sc_lookup_kernel(ea0_hbm, ea1_hbm, ea2_hbm, tab_hbm, out_hbm,
                      ea0_v, ea1_v, ea2_v, fidx_v, rows_v, sem):
    wid = lax.axis_index("s") * NC + lax.axis_index("c")
    ebase = wid * E_PER_W
    pltpu.sync_copy(ea0_hbm.at[pl.ds(ebase, E_PER_W)], ea0_v)
    pltpu.sync_copy(ea1_hbm.at[pl.ds(ebase, E_PER_W)], ea1_v)
    pltpu.sync_copy(ea2_hbm.at[pl.ds(ebase, E_PER_W)], ea2_v)

    def chunk_body(c, carry):
        for j in range(CHUNK // 16):
            off = c * CHUNK + j * 16
            i0 = ea0_v[pl.ds(off, 16)]
            i1 = ea1_v[pl.ds(off, 16)]
            i2 = ea2_v[pl.ds(off, 16)]
            fidx_v[pl.ds(j * 16, 16)] = i0 * (T1 * T2) + i1 * T2 + i2
        pltpu.async_copy(tab_hbm.at[fidx_v], rows_v, sem).wait()
        pltpu.sync_copy(rows_v, out_hbm.at[pl.ds(ebase + c * CHUNK, CHUNK)])
        return carry

    lax.fori_loop(0, NCHUNK, chunk_body, 0)


def kernel(edge_attr, W0, W1, W2):
    tab = _build_fused_table(W0, W1, W2)
    ea32 = edge_attr.astype(jnp.int32)
    ea0, ea1, ea2 = ea32[:, 0], ea32[:, 1], ea32[:, 2]

    mesh = plsc.VectorSubcoreMesh(core_axis_name="c", subcore_axis_name="s")
    run = functools.partial(
        pl.kernel,
        mesh=mesh,
        out_type=jax.ShapeDtypeStruct((E_TOTAL, EMB), jnp.float32),
        scratch_types=[
            pltpu.VMEM((E_PER_W,), jnp.int32),
            pltpu.VMEM((E_PER_W,), jnp.int32),
            pltpu.VMEM((E_PER_W,), jnp.int32),
            pltpu.VMEM((CHUNK,), jnp.int32),
            pltpu.VMEM((CHUNK, EMB), jnp.float32),
            pltpu.SemaphoreType.DMA,
        ],
    )(_sc_lookup_kernel)
    return run(ea0, ea1, ea2, tab)


# speedup vs baseline: 10.4954x; 1.4064x over previous
import functools

import numpy as np
import jax
import jax.numpy as jnp
from jax import lax
from jax.experimental import pallas as pl
from jax.experimental.pallas import tpu as pltpu
from jax.experimental.pallas import tpu_sc as plsc

EMB = 128
T0, T1, T2 = 15, 16, 12
NROWS = T0 * T1 * T2
E_TOTAL = 320000
NC, NS = 2, 16
NW = NC * NS
E_PER_W = E_TOTAL // NW
CHUNK = 80
NCHUNK = E_PER_W // CHUNK

_r = np.arange(NROWS)
_M = np.zeros((NROWS, 128), np.float32)
_M[_r, _r // (T1 * T2)] = 1.0
_M[_r, T0 + (_r // T2) % T1] = 1.0
_M[_r, T0 + T1 + _r % T2] = 1.0


def _fuse_body(m_ref, w_ref, out_ref):
    out_ref[...] = jnp.dot(m_ref[...], w_ref[...],
                           preferred_element_type=jnp.float32)


def _build_fused_table(W0, W1, W2):
    wcat = jnp.zeros((128, EMB), jnp.float32)
    wcat = lax.dynamic_update_slice(wcat, W0, (0, 0))
    wcat = lax.dynamic_update_slice(wcat, W1, (T0, 0))
    wcat = lax.dynamic_update_slice(wcat, W2, (T0 + T1, 0))
    return pl.pallas_call(
        _fuse_body,
        out_shape=jax.ShapeDtypeStruct((NROWS, EMB), jnp.float32),
    )(jnp.asarray(_M), wcat)


NBUF = 5


def _sc_lookup_kernel(ea0_hbm, ea1_hbm, ea2_hbm, tab_hbm, out_hbm,
                      ea0_v, ea1_v, ea2_v, fidx_v, rows_v, gsem, osem):
    wid = lax.axis_index("s") * NC + lax.axis_index("c")
    ebase = wid * E_PER_W
    pltpu.sync_copy(ea0_hbm.at[pl.ds(ebase, E_PER_W)], ea0_v)
    pltpu.sync_copy(ea1_hbm.at[pl.ds(ebase, E_PER_W)], ea1_v)
    pltpu.sync_copy(ea2_hbm.at[pl.ds(ebase, E_PER_W)], ea2_v)

    def start_gather(c, b):
        for j in range(CHUNK // 16):
            off = c * CHUNK + j * 16
            i0 = ea0_v[pl.ds(off, 16)]
            i1 = ea1_v[pl.ds(off, 16)]
            i2 = ea2_v[pl.ds(off, 16)]
            fidx_v[b, pl.ds(j * 16, 16)] = i0 * (T1 * T2) + i1 * T2 + i2
        pltpu.async_copy(tab_hbm.at[fidx_v.at[b]], rows_v.at[b], gsem.at[b])

    def start_out(c, b):
        pltpu.async_copy(rows_v.at[b], out_hbm.at[pl.ds(ebase + c * CHUNK, CHUNK)],
                         osem.at[b])

    def wait_gather(b):
        pltpu.make_async_copy(tab_hbm.at[fidx_v.at[b]], rows_v.at[b],
                              gsem.at[b]).wait()

    def wait_out(c, b):
        pltpu.make_async_copy(rows_v.at[b],
                              out_hbm.at[pl.ds(ebase + c * CHUNK, CHUNK)],
                              osem.at[b]).wait()

    for b in range(NBUF - 1):
        start_gather(b, b)

    def outer_body(k, carry):
        for b in range(NBUF):
            c = k * NBUF + b
            pb = (b + NBUF - 1) % NBUF
            if b == 0:
                @pl.when(k > 0)
                def _():
                    wait_out(c - 1, pb)
            else:
                wait_out(c - 1, pb)

            @pl.when(c + NBUF - 1 < NCHUNK)
            def _():
                start_gather(c + NBUF - 1, pb)

            wait_gather(b)
            start_out(c, b)
        return carry

    lax.fori_loop(0, NCHUNK // NBUF, outer_body, 0)
    wait_out(NCHUNK - 1, (NCHUNK - 1) % NBUF)


def kernel(edge_attr, W0, W1, W2):
    tab = _build_fused_table(W0, W1, W2)
    ea32 = edge_attr.astype(jnp.int32)
    ea0, ea1, ea2 = ea32[:, 0], ea32[:, 1], ea32[:, 2]

    mesh = plsc.VectorSubcoreMesh(core_axis_name="c", subcore_axis_name="s")
    run = functools.partial(
        pl.kernel,
        mesh=mesh,
        out_type=jax.ShapeDtypeStruct((E_TOTAL, EMB), jnp.float32),
        scratch_types=[
            pltpu.VMEM((E_PER_W,), jnp.int32),
            pltpu.VMEM((E_PER_W,), jnp.int32),
            pltpu.VMEM((E_PER_W,), jnp.int32),
            pltpu.VMEM((NBUF, CHUNK), jnp.int32),
            pltpu.VMEM((NBUF, CHUNK, EMB), jnp.float32),
            pltpu.SemaphoreType.DMA((NBUF,)),
            pltpu.SemaphoreType.DMA((NBUF,)),
        ],
    )(_sc_lookup_kernel)
    return run(ea0, ea1, ea2, tab)


# speedup vs baseline: 19.0281x; 1.8130x over previous
import functools

import numpy as np
import jax
import jax.numpy as jnp
from jax import lax
from jax.experimental import pallas as pl
from jax.experimental.pallas import tpu as pltpu
from jax.experimental.pallas import tpu_sc as plsc

EMB = 128
T0, T1, T2 = 15, 16, 12
NROWS = T0 * T1 * T2
E_TOTAL = 320000
NC, NS = 2, 16
NW = NC * NS
E_PER_W = E_TOTAL // NW
CHUNK = 80
NCHUNK = E_PER_W // CHUNK

_r = np.arange(NROWS)
_M = np.zeros((NROWS, 128), np.float32)
_M[_r, _r // (T1 * T2)] = 1.0
_M[_r, T0 + (_r // T2) % T1] = 1.0
_M[_r, T0 + T1 + _r % T2] = 1.0


def _fuse_body(m_ref, w_ref, out_ref):
    out_ref[...] = jnp.dot(m_ref[...], w_ref[...],
                           preferred_element_type=jnp.float32)


def _build_fused_table(W0, W1, W2):
    wcat = jnp.zeros((128, EMB), jnp.float32)
    wcat = lax.dynamic_update_slice(wcat, W0, (0, 0))
    wcat = lax.dynamic_update_slice(wcat, W1, (T0, 0))
    wcat = lax.dynamic_update_slice(wcat, W2, (T0 + T1, 0))
    return pl.pallas_call(
        _fuse_body,
        out_shape=jax.ShapeDtypeStruct((NROWS, EMB), jnp.float32),
    )(jnp.asarray(_M), wcat)


NBUF = 5


def _sc_lookup_kernel(ea0_hbm, ea1_hbm, ea2_hbm, tab_hbm, out_hbm,
                      ea0_v, ea1_v, ea2_v, fidx_v, rows_v, tab_sh, gsem, osem):
    wid = lax.axis_index("s") * NC + lax.axis_index("c")
    ebase = wid * E_PER_W
    @pl.when(lax.axis_index("s") == 0)
    def _():
        pltpu.sync_copy(tab_hbm, tab_sh)

    pltpu.sync_copy(ea0_hbm.at[pl.ds(ebase, E_PER_W)], ea0_v)
    pltpu.sync_copy(ea1_hbm.at[pl.ds(ebase, E_PER_W)], ea1_v)
    pltpu.sync_copy(ea2_hbm.at[pl.ds(ebase, E_PER_W)], ea2_v)
    plsc.subcore_barrier()

    def start_gather(c, b):
        for j in range(CHUNK // 16):
            off = c * CHUNK + j * 16
            i0 = ea0_v[pl.ds(off, 16)]
            i1 = ea1_v[pl.ds(off, 16)]
            i2 = ea2_v[pl.ds(off, 16)]
            fidx_v[b, pl.ds(j * 16, 16)] = i0 * (T1 * T2) + i1 * T2 + i2
        pltpu.async_copy(tab_sh.at[fidx_v.at[b]], rows_v.at[b], gsem.at[b])

    def start_out(c, b):
        pltpu.async_copy(rows_v.at[b], out_hbm.at[pl.ds(ebase + c * CHUNK, CHUNK)],
                         osem.at[b])

    def wait_gather(b):
        pltpu.make_async_copy(tab_sh.at[fidx_v.at[b]], rows_v.at[b],
                              gsem.at[b]).wait()

    def wait_out(c, b):
        pltpu.make_async_copy(rows_v.at[b],
                              out_hbm.at[pl.ds(ebase + c * CHUNK, CHUNK)],
                              osem.at[b]).wait()

    for b in range(NBUF - 1):
        start_gather(b, b)

    def outer_body(k, carry):
        for b in range(NBUF):
            c = k * NBUF + b
            pb = (b + NBUF - 1) % NBUF
            if b == 0:
                @pl.when(k > 0)
                def _():
                    wait_out(c - 1, pb)
            else:
                wait_out(c - 1, pb)

            @pl.when(c + NBUF - 1 < NCHUNK)
            def _():
                start_gather(c + NBUF - 1, pb)

            wait_gather(b)
            start_out(c, b)
        return carry

    lax.fori_loop(0, NCHUNK // NBUF, outer_body, 0)
    wait_out(NCHUNK - 1, (NCHUNK - 1) % NBUF)


def kernel(edge_attr, W0, W1, W2):
    tab = _build_fused_table(W0, W1, W2)
    ea32 = edge_attr.astype(jnp.int32)
    ea0, ea1, ea2 = ea32[:, 0], ea32[:, 1], ea32[:, 2]

    mesh = plsc.VectorSubcoreMesh(core_axis_name="c", subcore_axis_name="s")
    run = functools.partial(
        pl.kernel,
        mesh=mesh,
        out_type=jax.ShapeDtypeStruct((E_TOTAL, EMB), jnp.float32),
        scratch_types=[
            pltpu.VMEM((E_PER_W,), jnp.int32),
            pltpu.VMEM((E_PER_W,), jnp.int32),
            pltpu.VMEM((E_PER_W,), jnp.int32),
            pltpu.VMEM((NBUF, CHUNK), jnp.int32),
            pltpu.VMEM((NBUF, CHUNK, EMB), jnp.float32),
            pltpu.VMEM_SHARED((NROWS, EMB), jnp.float32),
            pltpu.SemaphoreType.DMA((NBUF,)),
            pltpu.SemaphoreType.DMA((NBUF,)),
        ],
    )(_sc_lookup_kernel)
    return run(ea0, ea1, ea2, tab)


# speedup vs baseline: 19.5325x; 1.0265x over previous
import functools

import numpy as np
import jax
import jax.numpy as jnp
from jax import lax
from jax.experimental import pallas as pl
from jax.experimental.pallas import tpu as pltpu
from jax.experimental.pallas import tpu_sc as plsc

EMB = 128
T0, T1, T2 = 15, 16, 12
NROWS = T0 * T1 * T2
E_TOTAL = 320000
NC, NS = 2, 16
NW = NC * NS
E_PER_W = E_TOTAL // NW
CHUNK = 80
NCHUNK = E_PER_W // CHUNK

_r = np.arange(NROWS)
_M = np.zeros((NROWS, 128), np.float32)
_M[_r, _r // (T1 * T2)] = 1.0
_M[_r, T0 + (_r // T2) % T1] = 1.0
_M[_r, T0 + T1 + _r % T2] = 1.0


def _fuse_body(m_ref, w_ref, out_ref):
    out_ref[...] = jnp.dot(m_ref[...], w_ref[...],
                           preferred_element_type=jnp.float32)


def _build_fused_table(W0, W1, W2):
    wcat = jnp.zeros((128, EMB), jnp.float32)
    wcat = lax.dynamic_update_slice(wcat, W0, (0, 0))
    wcat = lax.dynamic_update_slice(wcat, W1, (T0, 0))
    wcat = lax.dynamic_update_slice(wcat, W2, (T0 + T1, 0))
    return pl.pallas_call(
        _fuse_body,
        out_shape=jax.ShapeDtypeStruct((NROWS, EMB), jnp.float32),
    )(jnp.asarray(_M), wcat)


NBUF = 5


def _sc_lookup_kernel(ea0_hbm, ea1_hbm, ea2_hbm, tab_hbm, out_hbm,
                      ea0_v, ea1_v, ea2_v, fidx_v, rows_v, tab_sh, gsem, osem):
    sid = lax.axis_index("s")
    wid = sid * NC + lax.axis_index("c")
    ebase = wid * E_PER_W
    trows = 192
    toff = sid * trows

    @pl.when(sid < NROWS // trows)
    def _():
        pltpu.async_copy(tab_hbm.at[pl.ds(toff, trows)],
                         tab_sh.at[pl.ds(toff, trows)], gsem.at[1])

    pltpu.async_copy(ea0_hbm.at[pl.ds(ebase, E_PER_W)], ea0_v, gsem.at[0])
    pltpu.async_copy(ea1_hbm.at[pl.ds(ebase, E_PER_W)], ea1_v, gsem.at[0])
    pltpu.async_copy(ea2_hbm.at[pl.ds(ebase, E_PER_W)], ea2_v, gsem.at[0])
    pltpu.make_async_copy(ea0_hbm.at[pl.ds(ebase, E_PER_W)], ea0_v,
                          gsem.at[0]).wait()
    pltpu.make_async_copy(ea1_hbm.at[pl.ds(ebase, E_PER_W)], ea1_v,
                          gsem.at[0]).wait()
    pltpu.make_async_copy(ea2_hbm.at[pl.ds(ebase, E_PER_W)], ea2_v,
                          gsem.at[0]).wait()

    @pl.when(sid < NROWS // trows)
    def _():
        pltpu.make_async_copy(tab_hbm.at[pl.ds(toff, trows)],
                              tab_sh.at[pl.ds(toff, trows)], gsem.at[1]).wait()

    plsc.subcore_barrier()

    def start_gather(c, b):
        for j in range(CHUNK // 16):
            off = c * CHUNK + j * 16
            i0 = ea0_v[pl.ds(off, 16)]
            i1 = ea1_v[pl.ds(off, 16)]
            i2 = ea2_v[pl.ds(off, 16)]
            fidx_v[b, pl.ds(j * 16, 16)] = i0 * (T1 * T2) + i1 * T2 + i2
        pltpu.async_copy(tab_sh.at[fidx_v.at[b]], rows_v.at[b], gsem.at[b])

    def start_out(c, b):
        pltpu.async_copy(rows_v.at[b], out_hbm.at[pl.ds(ebase + c * CHUNK, CHUNK)],
                         osem.at[b])

    def wait_gather(b):
        pltpu.make_async_copy(tab_sh.at[fidx_v.at[b]], rows_v.at[b],
                              gsem.at[b]).wait()

    def wait_out(c, b):
        pltpu.make_async_copy(rows_v.at[b],
                              out_hbm.at[pl.ds(ebase + c * CHUNK, CHUNK)],
                              osem.at[b]).wait()

    for b in range(NBUF - 1):
        start_gather(b, b)

    def outer_body(k, carry):
        for b in range(NBUF):
            c = k * NBUF + b
            pb = (b + NBUF - 1) % NBUF
            if b == 0:
                @pl.when(k > 0)
                def _():
                    wait_out(c - 1, pb)
            else:
                wait_out(c - 1, pb)

            @pl.when(c + NBUF - 1 < NCHUNK)
            def _():
                start_gather(c + NBUF - 1, pb)

            wait_gather(b)
            start_out(c, b)
        return carry

    lax.fori_loop(0, NCHUNK // NBUF, outer_body, 0)
    wait_out(NCHUNK - 1, (NCHUNK - 1) % NBUF)


def kernel(edge_attr, W0, W1, W2):
    tab = _build_fused_table(W0, W1, W2)
    ea32 = edge_attr.astype(jnp.int32)
    ea0, ea1, ea2 = ea32[:, 0], ea32[:, 1], ea32[:, 2]

    mesh = plsc.VectorSubcoreMesh(core_axis_name="c", subcore_axis_name="s")
    run = functools.partial(
        pl.kernel,
        mesh=mesh,
        out_type=jax.ShapeDtypeStruct((E_TOTAL, EMB), jnp.float32),
        scratch_types=[
            pltpu.VMEM((E_PER_W,), jnp.int32),
            pltpu.VMEM((E_PER_W,), jnp.int32),
            pltpu.VMEM((E_PER_W,), jnp.int32),
            pltpu.VMEM((NBUF, CHUNK), jnp.int32),
            pltpu.VMEM((NBUF, CHUNK, EMB), jnp.float32),
            pltpu.VMEM_SHARED((NROWS, EMB), jnp.float32),
            pltpu.SemaphoreType.DMA((NBUF,)),
            pltpu.SemaphoreType.DMA((NBUF,)),
        ],
    )(_sc_lookup_kernel)
    return run(ea0, ea1, ea2, tab)
